# baseline (speedup 1.0000x reference)
import math

import jax
import jax.numpy as jnp
from jax.experimental import pallas as pl
from jax.experimental.pallas import tpu as pltpu

_PI = math.pi
_TWO_PI = 2.0 * math.pi
_HALF_PI = 0.5 * math.pi
_INV_TWO_PI = 1.0 / _TWO_PI

_ATAN_C = (
    0.9999966346599344,
    -0.3331830275252533,
    0.19813212106599729,
    -0.1324751723201036,
    0.07981110084304613,
    -0.033725845571015184,
    0.006842593618516107,
)


def _wrap(a):
    return a - _TWO_PI * jnp.floor((a + _PI) * _INV_TWO_PI)


def _atan2(y, x):
    ax = jnp.abs(x)
    ay = jnp.abs(y)
    hi = jnp.maximum(ax, ay)
    lo = jnp.minimum(ax, ay)
    a = lo / jnp.where(hi == 0.0, 1.0, hi)
    s = a * a
    p = jnp.float32(_ATAN_C[6])
    for c in (_ATAN_C[5], _ATAN_C[4], _ATAN_C[3], _ATAN_C[2], _ATAN_C[1],
              _ATAN_C[0]):
        p = p * s + jnp.float32(c)
    r = a * p
    r = jnp.where(ay > ax, _HALF_PI - r, r)
    r = jnp.where(x < 0.0, _PI - r, r)
    return jnp.where(y < 0.0, -r, r)


def _geom_kernel(prm_ref, ptxy_ref, opt_ref,
                 r_pl2pl_ref, r_pt2pl_ref, mask_ref):
    r = pl.program_id(1)
    R = r_pl2pl_ref.shape[2]
    base = r * R

    xj = prm_ref[0, 0, :]
    yj = prm_ref[0, 1, :]
    oj = prm_ref[0, 2, :]
    vj = prm_ref[0, 3, :]

    xi = prm_ref[0, 0, pl.ds(base, R)]
    yi = prm_ref[0, 1, pl.ds(base, R)]
    oi = prm_ref[0, 2, pl.ds(base, R)]
    vi = prm_ref[0, 3, pl.ds(base, R)]

    oi_col = oi[:, None]

    dx = xj[None, :] - xi[:, None]
    dy = yj[None, :] - yi[:, None]
    r_pl2pl_ref[0, 0, :, :] = jnp.sqrt(dx * dx + dy * dy)
    r_pl2pl_ref[0, 1, :, :] = _wrap(_atan2(dy, dx) - oi_col)
    r_pl2pl_ref[0, 2, :, :] = _wrap(oi_col - oj[None, :])

    n = xj.shape[0]
    row = jax.lax.broadcasted_iota(jnp.int32, (R, n), 0) + base
    col = jax.lax.broadcasted_iota(jnp.int32, (R, n), 1)
    mask_ref[0, :, :] = (vi[:, None] > 0.0) & (vj[None, :] > 0.0) & (row != col)

    dxp = ptxy_ref[0, :, 0, :] * 0.1 - xi[:, None]
    dyp = ptxy_ref[0, :, 1, :] * 0.1 - yi[:, None]
    r_pt2pl_ref[0, 0, :, :] = jnp.sqrt(dxp * dxp + dyp * dyp)
    r_pt2pl_ref[0, 1, :, :] = _wrap(_atan2(dyp, dxp) - oi_col)
    r_pt2pl_ref[0, 2, :, :] = _wrap(opt_ref[0, :, :] - oi_col)


def kernel(pos_pt, orient_pt, pos_pl, orient_pl, valid_pl):
    B, PL, PT, _ = pos_pt.shape
    R = 512

    ptxy = jnp.swapaxes(pos_pt, 2, 3)
    prm = jnp.stack(
        [pos_pl[..., 0] * 0.1, pos_pl[..., 1] * 0.1, orient_pl,
         valid_pl.astype(jnp.float32)], axis=1)

    prm_spec = pl.BlockSpec((1, 4, PL), lambda b, r: (b, 0, 0))
    ptxy_spec = pl.BlockSpec((1, R, 2, PT), lambda b, r: (b, r, 0, 0))
    opt_spec = pl.BlockSpec((1, R, PT), lambda b, r: (b, r, 0))

    r_pl2pl, r_pt2pl, mask = pl.pallas_call(
        _geom_kernel,
        grid=(B, PL // R),
        in_specs=[prm_spec, ptxy_spec, opt_spec],
        out_specs=[
            pl.BlockSpec((1, 3, R, PL), lambda b, r: (b, 0, r, 0)),
            pl.BlockSpec((1, 3, R, PT), lambda b, r: (b, 0, r, 0)),
            pl.BlockSpec((1, R, PL), lambda b, r: (b, r, 0)),
        ],
        out_shape=(
            jax.ShapeDtypeStruct((B, 3, PL, PL), jnp.float32),
            jax.ShapeDtypeStruct((B, 3, PL, PT), jnp.float32),
            jax.ShapeDtypeStruct((B, PL, PL), jnp.bool_),
        ),
        compiler_params=pltpu.CompilerParams(
            dimension_semantics=("parallel", "parallel"),
        ),
    )(prm, ptxy, orient_pt)

    return (r_pl2pl, r_pt2pl, mask)


# speedup vs baseline: 2.0465x; 2.0465x over previous
import math

import jax
import jax.numpy as jnp
from jax.experimental import pallas as pl
from jax.experimental.pallas import tpu as pltpu

_PI = math.pi
_TWO_PI = 2.0 * math.pi
_HALF_PI = 0.5 * math.pi
_INV_TWO_PI = 1.0 / _TWO_PI

_ATAN_C = (
    0.9999966346599344,
    -0.3331830275252533,
    0.19813212106599729,
    -0.1324751723201036,
    0.07981110084304613,
    -0.033725845571015184,
    0.006842593618516107,
)


def _wrap(a):
    return a - _TWO_PI * jnp.floor((a + _PI) * _INV_TWO_PI)


def _atan2(y, x):
    ax = jnp.abs(x)
    ay = jnp.abs(y)
    hi = jnp.maximum(ax, ay)
    lo = jnp.minimum(ax, ay)
    a = lo / jnp.where(hi == 0.0, 1.0, hi)
    s = a * a
    p = jnp.float32(_ATAN_C[6])
    for c in (_ATAN_C[5], _ATAN_C[4], _ATAN_C[3], _ATAN_C[2], _ATAN_C[1],
              _ATAN_C[0]):
        p = p * s + jnp.float32(c)
    r = a * p
    r = jnp.where(ay > ax, _HALF_PI - r, r)
    r = jnp.where(x < 0.0, _PI - r, r)
    return jnp.where(y < 0.0, -r, r)


def _geom_kernel(prm_ref, ptxy_ref, opt_ref,
                 r_pl2pl_ref, r_pt2pl_ref, mask_ref):
    r = pl.program_id(1)
    R = r_pl2pl_ref.shape[2]
    base = r * R

    xj = prm_ref[0, 0, :]
    yj = prm_ref[0, 1, :]
    oj = prm_ref[0, 2, :]
    vj = prm_ref[0, 3, :]

    xi = prm_ref[0, 0, pl.ds(base, R)]
    yi = prm_ref[0, 1, pl.ds(base, R)]
    oi = prm_ref[0, 2, pl.ds(base, R)]
    vi = prm_ref[0, 3, pl.ds(base, R)]

    oi_col = oi[:, None]

    dx = xj[None, :] - xi[:, None]
    dy = yj[None, :] - yi[:, None]
    r_pl2pl_ref[0, 0, :, :] = jnp.sqrt(dx * dx + dy * dy)
    r_pl2pl_ref[0, 1, :, :] = _wrap(_atan2(dy, dx) - oi_col)
    r_pl2pl_ref[0, 2, :, :] = _wrap(oi_col - oj[None, :])

    n = xj.shape[0]
    row = jax.lax.broadcasted_iota(jnp.int32, (R, n), 0) + base
    col = jax.lax.broadcasted_iota(jnp.int32, (R, n), 1)
    mask_ref[0, :, :] = (vi[:, None] > 0.0) & (vj[None, :] > 0.0) & (row != col)

    npt = opt_ref.shape[2]
    dxp = ptxy_ref[0, :, :npt] - xi[:, None]
    dyp = ptxy_ref[0, :, npt:] - yi[:, None]
    r_pt2pl_ref[0, 0, :, :] = jnp.sqrt(dxp * dxp + dyp * dyp)
    r_pt2pl_ref[0, 1, :, :] = _wrap(_atan2(dyp, dxp) - oi_col)
    r_pt2pl_ref[0, 2, :, :] = _wrap(opt_ref[0, :, :] - oi_col)


def kernel(pos_pt, orient_pt, pos_pl, orient_pl, valid_pl):
    B, PL, PT, _ = pos_pt.shape
    R = 512

    ptxy = jnp.concatenate([pos_pt[..., 0], pos_pt[..., 1]], axis=-1) * 0.1
    prm = jnp.stack(
        [pos_pl[..., 0] * 0.1, pos_pl[..., 1] * 0.1, orient_pl,
         valid_pl.astype(jnp.float32)], axis=1)

    prm_spec = pl.BlockSpec((1, 4, PL), lambda b, r: (b, 0, 0))
    ptxy_spec = pl.BlockSpec((1, R, 2 * PT), lambda b, r: (b, r, 0))
    opt_spec = pl.BlockSpec((1, R, PT), lambda b, r: (b, r, 0))

    r_pl2pl, r_pt2pl, mask = pl.pallas_call(
        _geom_kernel,
        grid=(B, PL // R),
        in_specs=[prm_spec, ptxy_spec, opt_spec],
        out_specs=[
            pl.BlockSpec((1, 3, R, PL), lambda b, r: (b, 0, r, 0)),
            pl.BlockSpec((1, 3, R, PT), lambda b, r: (b, 0, r, 0)),
            pl.BlockSpec((1, R, PL), lambda b, r: (b, r, 0)),
        ],
        out_shape=(
            jax.ShapeDtypeStruct((B, 3, PL, PL), jnp.float32),
            jax.ShapeDtypeStruct((B, 3, PL, PT), jnp.float32),
            jax.ShapeDtypeStruct((B, PL, PL), jnp.bool_),
        ),
        compiler_params=pltpu.CompilerParams(
            dimension_semantics=("parallel", "parallel"),
        ),
    )(prm, ptxy, orient_pt)

    return (r_pl2pl, r_pt2pl, mask)


# speedup vs baseline: 2.0906x; 1.0216x over previous
import math

import jax
import jax.numpy as jnp
from jax.experimental import pallas as pl
from jax.experimental.pallas import tpu as pltpu

_PI = math.pi
_TWO_PI = 2.0 * math.pi
_HALF_PI = 0.5 * math.pi
_INV_TWO_PI = 1.0 / _TWO_PI

_ATAN_C = (
    0.9999966346599344,
    -0.3331830275252533,
    0.19813212106599729,
    -0.1324751723201036,
    0.07981110084304613,
    -0.033725845571015184,
    0.006842593618516107,
)


def _wrap(a):
    return a - _TWO_PI * jnp.floor((a + _PI) * _INV_TWO_PI)


def _atan2(y, x):
    ax = jnp.abs(x)
    ay = jnp.abs(y)
    hi = jnp.maximum(ax, ay)
    lo = jnp.minimum(ax, ay)
    a = lo / jnp.where(hi == 0.0, 1.0, hi)
    s = a * a
    p = jnp.float32(_ATAN_C[6])
    for c in (_ATAN_C[5], _ATAN_C[4], _ATAN_C[3], _ATAN_C[2], _ATAN_C[1],
              _ATAN_C[0]):
        p = p * s + jnp.float32(c)
    r = a * p
    r = jnp.where(ay > ax, _HALF_PI - r, r)
    r = jnp.where(x < 0.0, _PI - r, r)
    return jnp.where(y < 0.0, -r, r)


def _geom_kernel(ptxy_ref, opt_ref,
                 r_pl2pl_ref, r_pt2pl_ref, mask_ref):
    n = r_pl2pl_ref.shape[3]

    def param(p):
        return jnp.concatenate(
            (ptxy_ref[0, n + 2 * p, :], ptxy_ref[0, n + 2 * p + 1, :]), axis=0)

    xj = param(0)
    yj = param(1)
    oj = param(2)
    vj = param(3)

    xi, yi, oi, vi = xj, yj, oj, vj
    oi_col = oi[:, None]

    dx = xj[None, :] - xi[:, None]
    dy = yj[None, :] - yi[:, None]
    r_pl2pl_ref[0, 0, :, :] = jnp.sqrt(dx * dx + dy * dy)
    r_pl2pl_ref[0, 1, :, :] = _wrap(_atan2(dy, dx) - oi_col)
    r_pl2pl_ref[0, 2, :, :] = _wrap(oi_col - oj[None, :])

    row = jax.lax.broadcasted_iota(jnp.int32, (n, n), 0)
    col = jax.lax.broadcasted_iota(jnp.int32, (n, n), 1)
    mask_ref[0, :, :] = (vi[:, None] > 0.0) & (vj[None, :] > 0.0) & (row != col)

    npt = opt_ref.shape[2]
    dxp = ptxy_ref[0, :n, :npt] - xi[:, None]
    dyp = ptxy_ref[0, :n, npt:] - yi[:, None]
    r_pt2pl_ref[0, 0, :, :] = jnp.sqrt(dxp * dxp + dyp * dyp)
    r_pt2pl_ref[0, 1, :, :] = _wrap(_atan2(dyp, dxp) - oi_col)
    r_pt2pl_ref[0, 2, :, :] = _wrap(opt_ref[0, :, :] - oi_col)


def kernel(pos_pt, orient_pt, pos_pl, orient_pl, valid_pl):
    B, PL, PT, _ = pos_pt.shape

    prm = jnp.stack(
        [pos_pl[..., 0] * 0.1, pos_pl[..., 1] * 0.1, orient_pl,
         valid_pl.astype(jnp.float32)], axis=1)
    ptxy = jnp.concatenate(
        [jnp.concatenate([pos_pt[..., 0], pos_pt[..., 1]], axis=-1) * 0.1,
         prm.reshape(B, 8, 2 * PT)], axis=1)

    ptxy_spec = pl.BlockSpec((1, PL + 8, 2 * PT), lambda b: (b, 0, 0))
    opt_spec = pl.BlockSpec((1, PL, PT), lambda b: (b, 0, 0))

    r_pl2pl, r_pt2pl, mask = pl.pallas_call(
        _geom_kernel,
        grid=(B,),
        in_specs=[ptxy_spec, opt_spec],
        out_specs=[
            pl.BlockSpec((1, 3, PL, PL), lambda b: (b, 0, 0, 0)),
            pl.BlockSpec((1, 3, PL, PT), lambda b: (b, 0, 0, 0)),
            pl.BlockSpec((1, PL, PL), lambda b: (b, 0, 0)),
        ],
        out_shape=(
            jax.ShapeDtypeStruct((B, 3, PL, PL), jnp.float32),
            jax.ShapeDtypeStruct((B, 3, PL, PT), jnp.float32),
            jax.ShapeDtypeStruct((B, PL, PL), jnp.bool_),
        ),
        compiler_params=pltpu.CompilerParams(
            dimension_semantics=("parallel",),
        ),
    )(ptxy, orient_pt)

    return (r_pl2pl, r_pt2pl, mask)
